# baseline (speedup 1.0000x reference)
import functools

import jax
import jax.numpy as jnp
from jax import lax
from jax.experimental import pallas as pl
from jax.experimental.pallas import tpu as pltpu
from jax.experimental.pallas import tpu_sc as plsc

N = 10000
G = 64
L = 3
P = 128
Q = 64
D_IN = 152
HW = 128
NP_ = 10240
SORT_N = 16384
E_PAD = 327680
CHUNK = 128
CPW = 80
NC, NS = 2, 16
ROWS_PT = NP_ // NS
SLABS = ROWS_PT // CHUNK
CB = 128
N_PASS = 105

f32 = jnp.float32
i32 = jnp.int32
MININT = -2147483648
MASK25 = (1 << 25) - 1


EPT = E_PAD // NS
CPT = EPT // CHUNK


SLABC = 16


def _sc_agg_body(h_hbm, src_hbm, dst_hbm, out_hbm, sslab, dslab, didx0,
                 didx1, gbuf0, gbuf1, agg_sp, gsem0, gsem1, ssem0, ssem1):
    c = lax.axis_index("c")
    s = lax.axis_index("s")
    ebase = s * EPT
    gbufs = (gbuf0, gbuf1)
    didxs = (didx0, didx1)
    gsems = (gsem0, gsem1)
    ssems = (ssem0, ssem1)

    def zrow(r, _):
        for k in range(HW // 16):
            gbuf0[r, pl.ds(16 * k, 16)] = jnp.zeros((16,), f32)
        return 0
    lax.fori_loop(0, CHUNK, zrow, 0)
    for slab in range(SLABS):
        pltpu.sync_copy(gbuf0, agg_sp.at[pl.ds(s * ROWS_PT + slab * CHUNK,
                                               CHUNK)])
    plsc.subcore_barrier()

    def group_body(g, _):
        gb = ebase + g * (SLABC * CHUNK)
        pltpu.sync_copy(src_hbm.at[pl.ds(gb, SLABC * CHUNK)], sslab)
        pltpu.sync_copy(dst_hbm.at[pl.ds(gb, SLABC * CHUNK)], dslab)
        for pp in range(SLABC // 2):
            gp = g * (SLABC // 2) + pp
            for b in (0, 1):
                lbase = (pp * 2 + b) * CHUNK

                @pl.when(gp > 0)
                def _free(b=b):
                    pltpu.make_async_copy(gbufs[b], agg_sp.at[didxs[b]],
                                          ssems[b]).wait()
                for kk in range(CHUNK // 16):
                    didxs[b][pl.ds(kk * 16, 16)] = (
                        dslab[pl.ds(lbase + kk * 16, 16)])
                pltpu.async_copy(
                    h_hbm.at[c].at[sslab.at[pl.ds(lbase, CHUNK)]],
                    gbufs[b], gsems[b])
            for b in (0, 1):
                lbase = (pp * 2 + b) * CHUNK
                pltpu.make_async_copy(
                    h_hbm.at[c].at[sslab.at[pl.ds(lbase, CHUNK)]],
                    gbufs[b], gsems[b]).wait()
                pltpu.async_copy(gbufs[b], agg_sp.at[didxs[b]], ssems[b],
                                 add=True)
        return 0
    lax.fori_loop(0, CPT // SLABC, group_body, 0)
    for b in (0, 1):
        pltpu.make_async_copy(gbufs[b], agg_sp.at[didxs[b]], ssems[b]).wait()
    plsc.subcore_barrier()

    for slab in range(SLABS):
        rows = pl.ds(s * ROWS_PT + slab * CHUNK, CHUNK)
        pltpu.sync_copy(agg_sp.at[rows], gbuf0)
        pltpu.sync_copy(gbuf0, out_hbm.at[c].at[rows])


@jax.jit
def _sc_agg(h, srcp, dstp):
    mesh = plsc.VectorSubcoreMesh(core_axis_name="c", subcore_axis_name="s",
                                  num_cores=NC, num_subcores=NS)
    return pl.kernel(
        _sc_agg_body,
        out_type=jax.ShapeDtypeStruct((NC, NP_, HW), f32),
        mesh=mesh,
        scratch_types=[
            pltpu.VMEM((SLABC * CHUNK,), i32),
            pltpu.VMEM((SLABC * CHUNK,), i32),
            pltpu.VMEM((CHUNK,), i32),
            pltpu.VMEM((CHUNK,), i32),
            pltpu.VMEM((CHUNK, HW), f32),
            pltpu.VMEM((CHUNK, HW), f32),
            pltpu.VMEM_SHARED((NP_, HW), f32),
            pltpu.SemaphoreType.DMA,
            pltpu.SemaphoreType.DMA,
            pltpu.SemaphoreType.DMA,
            pltpu.SemaphoreType.DMA,
        ],
    )(h, srcp, dstp)


_RB = 1280


def _keep_mask(shape):
    d0 = lax.broadcasted_iota(i32, shape, 0)
    lane = lax.broadcasted_iota(i32, shape, 2)
    return (d0 == 0) | (lane < 24)


def _step1_body(h_ref, a_ref, out_ref, dinv_ref):
    a = a_ref[...]
    dinv = 1.0 / jnp.maximum(a[1, :, 127:128], 1.0)
    h = 0.5 * (h_ref[...] + a * dinv)
    out_ref[...] = jnp.where(_keep_mask(h.shape), h, 0.0)
    dinv_ref[...] = dinv


def _stepl_body(h_ref, a_ref, dinv_ref, out_ref):
    a = a_ref[...]
    h = 0.5 * (h_ref[...] + a * dinv_ref[...])
    out_ref[...] = jnp.where(_keep_mask(h.shape), h, 0.0)


def _row_spec():
    return pl.BlockSpec((NC, _RB, HW), lambda i: (0, i, 0))


def _dinv_spec():
    return pl.BlockSpec((_RB, 1), lambda i: (i, 0))


@jax.jit
def _tc_step1(h, a):
    return pl.pallas_call(
        _step1_body,
        grid=(NP_ // _RB,),
        in_specs=[_row_spec(), _row_spec()],
        out_specs=[_row_spec(), _dinv_spec()],
        out_shape=[jax.ShapeDtypeStruct((NC, NP_, HW), f32),
                   jax.ShapeDtypeStruct((NP_, 1), f32)],
    )(h, a)


@jax.jit
def _tc_stepl(h, a, dinv):
    return pl.pallas_call(
        _stepl_body,
        grid=(NP_ // _RB,),
        in_specs=[_row_spec(), _row_spec(), _dinv_spec()],
        out_specs=_row_spec(),
        out_shape=jax.ShapeDtypeStruct((NC, NP_, HW), f32),
    )(h, a, dinv)


def _proj_body(h0_ref, h1_ref, h2_ref, h3_ref, tc_ref, tk_ref, bk_ref, out_ref):
    hs = [h0_ref[...], h1_ref[...], h2_ref[...], h3_ref[...]]
    acc = jnp.zeros((_RB, P), f32)
    for l, hl in enumerate(hs):
        acc = acc + lax.dot_general(
            hl[0], tc_ref[:, l * 128:(l + 1) * 128],
            (((1,), (1,)), ((), ())), preferred_element_type=f32)
    labs = []
    for blk_start, blk_w in ((0, 8), (8, 16)):
        for hl in hs:
            b = hl[1, :, blk_start:blk_start + blk_w]
            m = jnp.max(b, axis=-1, keepdims=True)
            ii = jnp.where(b == m,
                           lax.broadcasted_iota(i32, b.shape, 1), blk_w)
            labs.append(jnp.min(ii, axis=-1, keepdims=True).astype(f32))
    xcat = jnp.concatenate(labs, axis=-1)
    pcat = lax.dot_general(xcat, tk_ref[...], (((1,), (1,)), ((), ())),
                           preferred_element_type=f32)
    vals = jnp.concatenate([acc, pcat], axis=-1)
    vb = lax.bitcast_convert_type(vals, i32)
    u = jnp.where(vb < 0, ~vb, vb | MININT)
    comp = lax.shift_left(bk_ref[...], 25) | lax.shift_right_logical(u, 7)
    out_ref[...] = comp ^ MININT


@jax.jit
def _tc_proj(h0, h1, h2, h3, theta_cont, theta_cat, bk):
    return pl.pallas_call(
        _proj_body,
        grid=(NP_ // _RB,),
        in_specs=[_row_spec(), _row_spec(), _row_spec(), _row_spec(),
                  pl.BlockSpec((P, 4 * 128), lambda i: (0, 0)),
                  pl.BlockSpec((P, 8), lambda i: (0, 0)),
                  pl.BlockSpec((_RB, 1), lambda i: (i, 0))],
        out_specs=pl.BlockSpec((_RB, 2 * P), lambda i: (i, 0)),
        out_shape=jax.ShapeDtypeStruct((NP_, 2 * P), i32),
    )(h0, h1, h2, h3, theta_cont, theta_cat, bk)


_QR = 2048
_NQ = SORT_N // _QR
CB = 256


def _sort_body(d_ref, kk_ref, nq_ref, val_ref, out_ref):
    t = pl.program_id(0)

    @pl.when(t == 0)
    def _init():
        def icpy(q, _):
            rows = pl.ds(q * _QR, _QR)
            out_ref[rows, :] = val_ref[rows, :]
            return 0
        lax.fori_loop(0, _NQ, icpy, 0)

    d = d_ref[t]
    kk = kk_ref[t]

    def inquarter(q, _):
        base = pl.multiple_of(q * _QR, _QR)
        rows = pl.ds(base, _QR)
        k = out_ref[rows, :]
        ri = base + lax.broadcasted_iota(i32, (_QR, 1), 0)
        low = (ri & d) == 0
        desc = (ri & kk) != 0
        r = pltpu.roll(k, _QR - d, 0)
        take_low = (k > r) != desc
        lo_new = jnp.where(take_low, r, k)
        hi_new = jnp.where(take_low, k, r)
        out_ref[rows, :] = jnp.where(low, lo_new, pltpu.roll(hi_new, d, 0))
        return 0

    def crossquarter(tt, _):
        m = d // _QR
        lo = pl.multiple_of(((tt // m) * (2 * m) + (tt % m)) * _QR, _QR)
        rlo = pl.ds(lo, _QR)
        rhi = pl.ds(pl.multiple_of(lo + d, _QR), _QR)
        klo = out_ref[rlo, :]
        khi = out_ref[rhi, :]
        take = (klo > khi) != ((lo & kk) != 0)
        out_ref[rlo, :] = jnp.where(take, khi, klo)
        out_ref[rhi, :] = jnp.where(take, klo, khi)
        return 0

    lax.cond(d < _QR,
             lambda: lax.fori_loop(0, nq_ref[t], inquarter, 0),
             lambda: lax.fori_loop(0, _NQ // 2, crossquarter, 0))


@jax.jit
def _tc_sort(val):
    ds, kks, nqs = [], [], []
    for st in range(14):
        for j in range(st, -1, -1):
            ds.append(1 << j)
            kks.append(1 << (st + 1))
            nqs.append(5 if st <= 10 else (6 if st == 11 else 8))
    grid_spec = pltpu.PrefetchScalarGridSpec(
        num_scalar_prefetch=3,
        grid=(N_PASS,),
        in_specs=[pl.BlockSpec((SORT_N, CB), lambda t, d, kk, nq: (0, 0))],
        out_specs=pl.BlockSpec((SORT_N, CB), lambda t, d, kk, nq: (0, 0)),
    )
    return pl.pallas_call(
        _sort_body,
        grid_spec=grid_spec,
        out_shape=jax.ShapeDtypeStruct((SORT_N, CB), i32),
    )(jnp.asarray(ds, i32), jnp.asarray(kks, i32), jnp.asarray(nqs, i32), val)


def _qidx_body(key_ref, li_ref, hi_ref, lw_ref, hw_ref):
    giota = lax.broadcasted_iota(i32, (1, G), 1)

    def cbody(j, acc):
        chunk = key_ref[pl.ds(j * 1024, 1024), :]
        eq = (chunk == giota).astype(f32)
        return acc + jnp.sum(eq, axis=0, keepdims=True)
    counts = lax.fori_loop(0, SORT_N // 1024, cbody, jnp.zeros((1, G), f32))

    lt = (lax.broadcasted_iota(i32, (G, G), 0) <
          lax.broadcasted_iota(i32, (G, G), 1)).astype(f32)
    starts = lax.dot_general(counts, lt, (((1,), (0,)), ((), ())),
                             preferred_element_type=f32)
    delta = f32(1.0) / f32(Q - 1)
    qs = lax.broadcasted_iota(i32, (Q, 1), 0).astype(f32) * delta
    pos = qs * (counts - 1.0)
    lo = jnp.floor(pos)
    hi = jnp.ceil(pos)
    hw = pos - lo
    lw_ref[...] = 1.0 - hw
    hw_ref[...] = hw
    li_ref[...] = (jnp.clip(lo, 0.0, counts - 1.0) + starts).astype(i32)
    hi_ref[...] = (jnp.clip(hi, 0.0, counts - 1.0) + starts).astype(i32)


@jax.jit
def _tc_qidx(key):
    return pl.pallas_call(
        _qidx_body,
        in_specs=[pl.BlockSpec((SORT_N, G), lambda: (0, 0))],
        out_specs=[pl.BlockSpec((Q, G), lambda: (0, 0))] * 4,
        out_shape=[jax.ShapeDtypeStruct((Q, G), i32),
                   jax.ShapeDtypeStruct((Q, G), i32),
                   jax.ShapeDtypeStruct((Q, G), f32),
                   jax.ShapeDtypeStruct((Q, G), f32)],
    )(key)


def _qgather_body(li_ref, hi_ref, lw_ref, hw_ref, srt_ref, out_ref, scr):
    g = pl.program_id(0)

    def decode(krow):
        comp = krow ^ MININT
        u = lax.shift_left(comp & MASK25, 7)
        vb = jnp.where(u < 0, u & jnp.int32(0x7FFFFFFF), ~u)
        return lax.bitcast_convert_type(vb, f32)

    def qbody(q, _):
        li = li_ref[q, g]
        hi = hi_ref[q, g]
        lw = lw_ref[q, g]
        hw = hw_ref[q, g]
        row = (decode(srt_ref[pl.ds(li, 1), :]) * lw +
               decode(srt_ref[pl.ds(hi, 1), :]) * hw)
        scr[pl.ds(q, 1), :] = row
        return 0
    lax.fori_loop(0, Q, qbody, 0)
    out_ref[0] = jnp.transpose(scr[...])


@jax.jit
def _tc_qgather(li, hi, lw, hw, srt):
    smem = pl.BlockSpec(memory_space=pltpu.SMEM)
    return pl.pallas_call(
        _qgather_body,
        grid=(G,),
        in_specs=[smem, smem, smem, smem,
                  pl.BlockSpec((SORT_N, 256), lambda g: (0, 0))],
        out_specs=pl.BlockSpec((1, 256, Q), lambda g: (g, 0, 0)),
        out_shape=jax.ShapeDtypeStruct((G, 256, Q), f32),
        scratch_shapes=[pltpu.VMEM((Q, 256), f32)],
    )(li, hi, lw, hw, srt)


def kernel(x, edge_index, batch, theta_cont, theta_cat):
    xf = x.astype(f32)
    half0 = jnp.zeros((NP_, HW), f32).at[:N].set(xf[:, :128])
    half1 = jnp.zeros((NP_, HW), f32)
    half1 = half1.at[:N, 0:24].set(xf[:, 128:152])
    half1 = half1.at[:N, HW - 1].set(1.0)
    h0 = jnp.stack([half0, half1])
    src = edge_index[0].astype(i32)
    dst = edge_index[1].astype(i32)
    e = src.shape[0]
    pad_idx = (N + (jnp.arange(E_PAD - e) % 16)).astype(i32)
    srcp = jnp.concatenate([src, pad_idx])
    dstp = jnp.concatenate([dst, pad_idx])

    agg = _sc_agg(h0, srcp, dstp)
    h1, dinv = _tc_step1(h0, agg)
    agg = _sc_agg(h1, srcp, dstp)
    h2 = _tc_stepl(h1, agg, dinv)
    agg = _sc_agg(h2, srcp, dstp)
    h3 = _tc_stepl(h2, agg, dinv)

    bk = jnp.full((NP_, 1), G, i32).at[:N, 0].set(batch.astype(i32))
    ks = _tc_proj(h0, h1, h2, h3, theta_cont, theta_cat, bk)

    val = jnp.full((SORT_N, 256), jnp.iinfo(i32).max, i32).at[:NP_].set(ks)
    srt = _tc_sort(val)

    key = jnp.full((SORT_N, 1), jnp.iinfo(i32).max, i32)
    key = key.at[:N, 0].set(batch.astype(i32))
    li, hi, lw, hw = _tc_qidx(jnp.broadcast_to(key, (SORT_N, G)))
    quant = _tc_qgather(li, hi, lw, hw, srt)
    return quant.reshape(G, 2 * P * Q)


# speedup vs baseline: 1.1322x; 1.1322x over previous
import functools

import jax
import jax.numpy as jnp
from jax import lax
from jax.experimental import pallas as pl
from jax.experimental.pallas import tpu as pltpu
from jax.experimental.pallas import tpu_sc as plsc

N = 10000
G = 64
L = 3
P = 128
Q = 64
D_IN = 152
HW = 128
NP_ = 10240
SORT_N = 16384
E_PAD = 327680
CHUNK = 128
CPW = 80
NC, NS = 2, 16
ROWS_PT = NP_ // NS
SLABS = ROWS_PT // CHUNK
CB = 128
N_PASS = 105

f32 = jnp.float32
i32 = jnp.int32
MININT = -2147483648
MASK25 = (1 << 25) - 1


EPT = E_PAD // NS
CPT = EPT // CHUNK


SLABC = 16


def _sc_agg_body(h_hbm, src_hbm, dst_hbm, out_hbm, sslab, dslab, didx0,
                 didx1, gbuf0, gbuf1, agg_sp, gsem0, gsem1, ssem0, ssem1):
    c = lax.axis_index("c")
    s = lax.axis_index("s")
    ebase = s * EPT
    gbufs = (gbuf0, gbuf1)
    didxs = (didx0, didx1)
    gsems = (gsem0, gsem1)
    ssems = (ssem0, ssem1)

    def zrow(r, _):
        for k in range(HW // 16):
            gbuf0[r, pl.ds(16 * k, 16)] = jnp.zeros((16,), f32)
        return 0
    lax.fori_loop(0, CHUNK, zrow, 0)
    for slab in range(SLABS):
        pltpu.sync_copy(gbuf0, agg_sp.at[pl.ds(s * ROWS_PT + slab * CHUNK,
                                               CHUNK)])
    plsc.subcore_barrier()

    def group_body(g, _):
        gb = ebase + g * (SLABC * CHUNK)
        pltpu.sync_copy(src_hbm.at[pl.ds(gb, SLABC * CHUNK)], sslab)
        pltpu.sync_copy(dst_hbm.at[pl.ds(gb, SLABC * CHUNK)], dslab)
        for pp in range(SLABC // 2):
            gp = g * (SLABC // 2) + pp
            for b in (0, 1):
                lbase = (pp * 2 + b) * CHUNK

                @pl.when(gp > 0)
                def _free(b=b):
                    pltpu.make_async_copy(gbufs[b], agg_sp.at[didxs[b]],
                                          ssems[b]).wait()
                for kk in range(CHUNK // 16):
                    didxs[b][pl.ds(kk * 16, 16)] = (
                        dslab[pl.ds(lbase + kk * 16, 16)])
                pltpu.async_copy(
                    h_hbm.at[c].at[sslab.at[pl.ds(lbase, CHUNK)]],
                    gbufs[b], gsems[b])
            for b in (0, 1):
                lbase = (pp * 2 + b) * CHUNK
                pltpu.make_async_copy(
                    h_hbm.at[c].at[sslab.at[pl.ds(lbase, CHUNK)]],
                    gbufs[b], gsems[b]).wait()
                pltpu.async_copy(gbufs[b], agg_sp.at[didxs[b]], ssems[b],
                                 add=True)
        return 0
    lax.fori_loop(0, CPT // SLABC, group_body, 0)
    for b in (0, 1):
        pltpu.make_async_copy(gbufs[b], agg_sp.at[didxs[b]], ssems[b]).wait()
    plsc.subcore_barrier()

    for slab in range(SLABS):
        rows = pl.ds(s * ROWS_PT + slab * CHUNK, CHUNK)
        pltpu.sync_copy(agg_sp.at[rows], gbuf0)
        pltpu.sync_copy(gbuf0, out_hbm.at[c].at[rows])


@jax.jit
def _sc_agg(h, srcp, dstp):
    mesh = plsc.VectorSubcoreMesh(core_axis_name="c", subcore_axis_name="s",
                                  num_cores=NC, num_subcores=NS)
    return pl.kernel(
        _sc_agg_body,
        out_type=jax.ShapeDtypeStruct((NC, NP_, HW), f32),
        mesh=mesh,
        scratch_types=[
            pltpu.VMEM((SLABC * CHUNK,), i32),
            pltpu.VMEM((SLABC * CHUNK,), i32),
            pltpu.VMEM((CHUNK,), i32),
            pltpu.VMEM((CHUNK,), i32),
            pltpu.VMEM((CHUNK, HW), f32),
            pltpu.VMEM((CHUNK, HW), f32),
            pltpu.VMEM_SHARED((NP_, HW), f32),
            pltpu.SemaphoreType.DMA,
            pltpu.SemaphoreType.DMA,
            pltpu.SemaphoreType.DMA,
            pltpu.SemaphoreType.DMA,
        ],
    )(h, srcp, dstp)


_RB = 1280


def _keep_mask(shape):
    d0 = lax.broadcasted_iota(i32, shape, 0)
    lane = lax.broadcasted_iota(i32, shape, 2)
    return (d0 == 0) | (lane < 24)


def _step1_body(h_ref, a_ref, out_ref, dinv_ref):
    a = a_ref[...]
    dinv = 1.0 / jnp.maximum(a[1, :, 127:128], 1.0)
    h = 0.5 * (h_ref[...] + a * dinv)
    out_ref[...] = jnp.where(_keep_mask(h.shape), h, 0.0)
    dinv_ref[...] = dinv


def _stepl_body(h_ref, a_ref, dinv_ref, out_ref):
    a = a_ref[...]
    h = 0.5 * (h_ref[...] + a * dinv_ref[...])
    out_ref[...] = jnp.where(_keep_mask(h.shape), h, 0.0)


def _row_spec():
    return pl.BlockSpec((NC, _RB, HW), lambda i: (0, i, 0))


def _dinv_spec():
    return pl.BlockSpec((_RB, 1), lambda i: (i, 0))


@jax.jit
def _tc_step1(h, a):
    return pl.pallas_call(
        _step1_body,
        grid=(NP_ // _RB,),
        in_specs=[_row_spec(), _row_spec()],
        out_specs=[_row_spec(), _dinv_spec()],
        out_shape=[jax.ShapeDtypeStruct((NC, NP_, HW), f32),
                   jax.ShapeDtypeStruct((NP_, 1), f32)],
    )(h, a)


@jax.jit
def _tc_stepl(h, a, dinv):
    return pl.pallas_call(
        _stepl_body,
        grid=(NP_ // _RB,),
        in_specs=[_row_spec(), _row_spec(), _dinv_spec()],
        out_specs=_row_spec(),
        out_shape=jax.ShapeDtypeStruct((NC, NP_, HW), f32),
    )(h, a, dinv)


def _proj_body(h0_ref, h1_ref, h2_ref, h3_ref, tc_ref, tk_ref, bk_ref, out_ref):
    hs = [h0_ref[...], h1_ref[...], h2_ref[...], h3_ref[...]]
    acc = jnp.zeros((_RB, P), f32)
    for l, hl in enumerate(hs):
        acc = acc + lax.dot_general(
            hl[0], tc_ref[:, l * 128:(l + 1) * 128],
            (((1,), (1,)), ((), ())), preferred_element_type=f32)
    labs = []
    for blk_start, blk_w in ((0, 8), (8, 16)):
        for hl in hs:
            b = hl[1, :, blk_start:blk_start + blk_w]
            m = jnp.max(b, axis=-1, keepdims=True)
            ii = jnp.where(b == m,
                           lax.broadcasted_iota(i32, b.shape, 1), blk_w)
            labs.append(jnp.min(ii, axis=-1, keepdims=True).astype(f32))
    xcat = jnp.concatenate(labs, axis=-1)
    pcat = lax.dot_general(xcat, tk_ref[...], (((1,), (1,)), ((), ())),
                           preferred_element_type=f32)
    vals = jnp.concatenate([acc, pcat], axis=-1)
    vb = lax.bitcast_convert_type(vals, i32)
    u = jnp.where(vb < 0, ~vb, vb | MININT)
    comp = lax.shift_left(bk_ref[...], 25) | lax.shift_right_logical(u, 7)
    out_ref[...] = comp ^ MININT


@jax.jit
def _tc_proj(h0, h1, h2, h3, theta_cont, theta_cat, bk):
    return pl.pallas_call(
        _proj_body,
        grid=(NP_ // _RB,),
        in_specs=[_row_spec(), _row_spec(), _row_spec(), _row_spec(),
                  pl.BlockSpec((P, 4 * 128), lambda i: (0, 0)),
                  pl.BlockSpec((P, 8), lambda i: (0, 0)),
                  pl.BlockSpec((_RB, 1), lambda i: (i, 0))],
        out_specs=pl.BlockSpec((_RB, 2 * P), lambda i: (i, 0)),
        out_shape=jax.ShapeDtypeStruct((NP_, 2 * P), i32),
    )(h0, h1, h2, h3, theta_cont, theta_cat, bk)


_QR = 2048
_NQ = SORT_N // _QR
CB = 256


def _sort_body(d_ref, kk_ref, nq_ref, val_ref, out_ref):
    t = pl.program_id(0)

    @pl.when(t == 0)
    def _init():
        def icpy(q, _):
            rows = pl.ds(q * _QR, _QR)
            out_ref[rows, :] = val_ref[rows, :]
            return 0
        lax.fori_loop(0, _NQ, icpy, 0)

    d = d_ref[t]
    kk = kk_ref[t]

    def inquarter(q, _):
        base = pl.multiple_of(q * _QR, _QR)
        rows = pl.ds(base, _QR)
        k = out_ref[rows, :]
        ri = base + lax.broadcasted_iota(i32, (_QR, 1), 0)
        low = (ri & d) == 0
        desc = (ri & kk) != 0
        want_min = low != desc
        pk = jnp.where(low, pltpu.roll(k, _QR - d, 0), pltpu.roll(k, d, 0))
        take = want_min == (k > pk)
        out_ref[rows, :] = jnp.where(take, pk, k)
        return 0

    def crossquarter(tt, _):
        m = d // _QR
        lo = pl.multiple_of(((tt // m) * (2 * m) + (tt % m)) * _QR, _QR)
        rlo = pl.ds(lo, _QR)
        rhi = pl.ds(pl.multiple_of(lo + d, _QR), _QR)
        klo = out_ref[rlo, :]
        khi = out_ref[rhi, :]
        take = (klo > khi) != ((lo & kk) != 0)
        out_ref[rlo, :] = jnp.where(take, khi, klo)
        out_ref[rhi, :] = jnp.where(take, klo, khi)
        return 0

    lax.cond(d < _QR,
             lambda: lax.fori_loop(0, nq_ref[t], inquarter, 0),
             lambda: lax.fori_loop(0, _NQ // 2, crossquarter, 0))


@jax.jit
def _tc_sort(val):
    ds, kks, nqs = [], [], []
    for st in range(14):
        for j in range(st, -1, -1):
            ds.append(1 << j)
            kks.append(1 << (st + 1))
            nqs.append(5 if st <= 10 else (6 if st == 11 else 8))
    grid_spec = pltpu.PrefetchScalarGridSpec(
        num_scalar_prefetch=3,
        grid=(N_PASS,),
        in_specs=[pl.BlockSpec((SORT_N, CB), lambda t, d, kk, nq: (0, 0))],
        out_specs=pl.BlockSpec((SORT_N, CB), lambda t, d, kk, nq: (0, 0)),
    )
    return pl.pallas_call(
        _sort_body,
        grid_spec=grid_spec,
        out_shape=jax.ShapeDtypeStruct((SORT_N, CB), i32),
    )(jnp.asarray(ds, i32), jnp.asarray(kks, i32), jnp.asarray(nqs, i32), val)


def _qidx_body(key_ref, li_ref, hi_ref, lw_ref, hw_ref):
    giota = lax.broadcasted_iota(i32, (1, G), 1)

    def cbody(j, acc):
        chunk = key_ref[pl.ds(j * 1024, 1024), :]
        eq = (chunk == giota).astype(f32)
        return acc + jnp.sum(eq, axis=0, keepdims=True)
    counts = lax.fori_loop(0, SORT_N // 1024, cbody, jnp.zeros((1, G), f32))

    lt = (lax.broadcasted_iota(i32, (G, G), 0) <
          lax.broadcasted_iota(i32, (G, G), 1)).astype(f32)
    starts = lax.dot_general(counts, lt, (((1,), (0,)), ((), ())),
                             preferred_element_type=f32)
    delta = f32(1.0) / f32(Q - 1)
    qs = lax.broadcasted_iota(i32, (Q, 1), 0).astype(f32) * delta
    pos = qs * (counts - 1.0)
    lo = jnp.floor(pos)
    hi = jnp.ceil(pos)
    hw = pos - lo
    lw_ref[...] = 1.0 - hw
    hw_ref[...] = hw
    li_ref[...] = (jnp.clip(lo, 0.0, counts - 1.0) + starts).astype(i32)
    hi_ref[...] = (jnp.clip(hi, 0.0, counts - 1.0) + starts).astype(i32)


@jax.jit
def _tc_qidx(key):
    return pl.pallas_call(
        _qidx_body,
        in_specs=[pl.BlockSpec((SORT_N, G), lambda: (0, 0))],
        out_specs=[pl.BlockSpec((Q, G), lambda: (0, 0))] * 4,
        out_shape=[jax.ShapeDtypeStruct((Q, G), i32),
                   jax.ShapeDtypeStruct((Q, G), i32),
                   jax.ShapeDtypeStruct((Q, G), f32),
                   jax.ShapeDtypeStruct((Q, G), f32)],
    )(key)


def _qgather_body(li_ref, hi_ref, lw_ref, hw_ref, srt_ref, out_ref, scr):
    g = pl.program_id(0)

    def decode(krow):
        comp = krow ^ MININT
        u = lax.shift_left(comp & MASK25, 7)
        vb = jnp.where(u < 0, u & jnp.int32(0x7FFFFFFF), ~u)
        return lax.bitcast_convert_type(vb, f32)

    def qbody(q, _):
        li = li_ref[q, g]
        hi = hi_ref[q, g]
        lw = lw_ref[q, g]
        hw = hw_ref[q, g]
        row = (decode(srt_ref[pl.ds(li, 1), :]) * lw +
               decode(srt_ref[pl.ds(hi, 1), :]) * hw)
        scr[pl.ds(q, 1), :] = row
        return 0
    lax.fori_loop(0, Q, qbody, 0)
    out_ref[0] = jnp.transpose(scr[...])


@jax.jit
def _tc_qgather(li, hi, lw, hw, srt):
    smem = pl.BlockSpec(memory_space=pltpu.SMEM)
    return pl.pallas_call(
        _qgather_body,
        grid=(G,),
        in_specs=[smem, smem, smem, smem,
                  pl.BlockSpec((SORT_N, 256), lambda g: (0, 0))],
        out_specs=pl.BlockSpec((1, 256, Q), lambda g: (g, 0, 0)),
        out_shape=jax.ShapeDtypeStruct((G, 256, Q), f32),
        scratch_shapes=[pltpu.VMEM((Q, 256), f32)],
    )(li, hi, lw, hw, srt)


def kernel(x, edge_index, batch, theta_cont, theta_cat):
    xf = x.astype(f32)
    half0 = jnp.zeros((NP_, HW), f32).at[:N].set(xf[:, :128])
    half1 = jnp.zeros((NP_, HW), f32)
    half1 = half1.at[:N, 0:24].set(xf[:, 128:152])
    half1 = half1.at[:N, HW - 1].set(1.0)
    h0 = jnp.stack([half0, half1])
    src = edge_index[0].astype(i32)
    dst = edge_index[1].astype(i32)
    e = src.shape[0]
    pad_idx = (N + (jnp.arange(E_PAD - e) % 16)).astype(i32)
    srcp = jnp.concatenate([src, pad_idx])
    dstp = jnp.concatenate([dst, pad_idx])

    agg = _sc_agg(h0, srcp, dstp)
    h1, dinv = _tc_step1(h0, agg)
    agg = _sc_agg(h1, srcp, dstp)
    h2 = _tc_stepl(h1, agg, dinv)
    agg = _sc_agg(h2, srcp, dstp)
    h3 = _tc_stepl(h2, agg, dinv)

    bk = jnp.full((NP_, 1), G, i32).at[:N, 0].set(batch.astype(i32))
    ks = _tc_proj(h0, h1, h2, h3, theta_cont, theta_cat, bk)

    val = jnp.full((SORT_N, 256), jnp.iinfo(i32).max, i32).at[:NP_].set(ks)
    srt = _tc_sort(val)

    key = jnp.full((SORT_N, 1), jnp.iinfo(i32).max, i32)
    key = key.at[:N, 0].set(batch.astype(i32))
    li, hi, lw, hw = _tc_qidx(jnp.broadcast_to(key, (SORT_N, G)))
    quant = _tc_qgather(li, hi, lw, hw, srt)
    return quant.reshape(G, 2 * P * Q)
